# baseline (speedup 1.0000x reference)
import jax
import jax.numpy as jnp
from jax.experimental import pallas as pl
from jax.experimental.pallas import tpu as pltpu

NB_CODE = 8192
CODE_DIM = 256
ATTN_DIM = 256
ENT_T = 0.01
EPS = 1e-05

_R = 256
_ROWS = 32 * 576
_GRID = _ROWS // _R
_SCALE = float(ATTN_DIM) ** -0.5


def _keys_kernel(cb_ref, wk_ref, kn_ref):
    k = jnp.dot(cb_ref[...], wk_ref[...], preferred_element_type=jnp.float32,
                precision=None)
    kn = k / (jnp.sqrt(jnp.sum(k * k, axis=1, keepdims=True)) + 1e-6)
    kn_ref[...] = kn * _SCALE


def _fused_kernel(xf_ref, cb_ref, kn_ref, wq_ref,
                  z_ref, ent_ref, vq_ref, ppl_ref, sent_ref, aent_ref,
                  psum_ref, csum_ref, sent_acc_ref, vq_acc_ref):
    i = pl.program_id(0)

    @pl.when(i == 0)
    def _init():
        psum_ref[...] = jnp.zeros_like(psum_ref)
        csum_ref[...] = jnp.zeros_like(csum_ref)
        sent_acc_ref[...] = jnp.zeros_like(sent_acc_ref)
        vq_acc_ref[...] = jnp.zeros_like(vq_acc_ref)

    xb = xf_ref[...]
    q = jnp.dot(xb, wq_ref[...], preferred_element_type=jnp.float32,
                precision=None)
    qn = q / (jnp.sqrt(jnp.sum(q * q, axis=1, keepdims=True)) + 1e-6)
    s = jax.lax.dot_general(qn, kn_ref[...], (((1,), (1,)), ((), ())),
                            preferred_element_type=jnp.float32,
                            precision=None)
    f32 = jnp.float32
    _sub = _R // 8
    _colsum8 = lambda a: jnp.sum(a.reshape(_sub, 8, NB_CODE), axis=0)

    e = jnp.exp(s)
    zd = jnp.dot(e, cb_ref[...], preferred_element_type=jnp.float32,
                 precision=None)
    zq = zd[:, :CODE_DIM] / zd[:, CODE_DIM:CODE_DIM + 1]
    d = zq - xb
    z_ref[...] = xb + d
    vq_acc_ref[...] += jnp.sum(d * d, axis=0, keepdims=True)

    e2 = jnp.exp(s * (1.0 / ENT_T))
    z2 = jnp.sum(e2, axis=1, keepdims=True)
    ses = jnp.sum(e2 * s, axis=1, keepdims=True)
    r2 = 1.0 / z2
    sent_acc_ref[...] += ses * (r2 * (1.0 / ENT_T)) - jnp.log(z2)
    psum_ref[...] += _colsum8(e2 * r2)

    m = jnp.max(s, axis=1, keepdims=True)
    oh = (s == m).astype(f32)
    colsum = _colsum8(oh)
    any_tie = jnp.sum(colsum) > _R + 0.5

    @pl.when(jnp.logical_not(any_tie))
    def _fast():
        csum_ref[...] += colsum

    @pl.when(any_tie)
    def _exact():
        iota = jax.lax.broadcasted_iota(jnp.int32, s.shape, 1)
        idx = jnp.min(jnp.where(s == m, iota, NB_CODE), axis=1,
                      keepdims=True)
        csum_ref[...] += _colsum8((iota == idx).astype(f32))

    @pl.when(i == _GRID - 1)
    def _fin():
        nrows = float(_ROWS)
        avg_p = jnp.sum(psum_ref[...], axis=0, keepdims=True) / nrows
        aent = -jnp.sum(avg_p * jnp.log(avg_p + EPS))
        sent = -jnp.sum(sent_acc_ref[...]) / nrows
        counts = jnp.sum(csum_ref[...], axis=0, keepdims=True)
        prob = counts / jnp.sum(counts)
        ppl = jnp.exp(-jnp.sum(prob * jnp.log(prob + 1e-7)))
        vq = jnp.sum(vq_acc_ref[...]) / (nrows * CODE_DIM)

        def bcast(v):
            return jnp.broadcast_to(jnp.reshape(v, (1, 1)), (1, 128))
        ent_ref[...] = bcast(sent - aent)
        vq_ref[...] = bcast(vq)
        ppl_ref[...] = bcast(ppl)
        sent_ref[...] = bcast(sent)
        aent_ref[...] = bcast(aent)


def _run(xf, codebook, Wq, Wk, interpret=False):
    f32 = jnp.float32
    scal = jax.ShapeDtypeStruct((1, 128), f32)
    const_spec = lambda shape: pl.BlockSpec(shape, lambda i: (0, 0))
    kn = pl.pallas_call(
        _keys_kernel,
        grid=(8,),
        in_specs=[
            pl.BlockSpec((NB_CODE // 8, CODE_DIM), lambda i: (i, 0)),
            pl.BlockSpec((CODE_DIM, ATTN_DIM), lambda i: (0, 0)),
        ],
        out_specs=pl.BlockSpec((NB_CODE // 8, ATTN_DIM), lambda i: (i, 0)),
        out_shape=jax.ShapeDtypeStruct((NB_CODE, ATTN_DIM), f32),
        interpret=interpret,
    )(codebook, Wk)
    cb_aug = jnp.concatenate(
        [codebook, jnp.ones((NB_CODE, 1), f32),
         jnp.zeros((NB_CODE, 127), f32)], axis=1)
    return pl.pallas_call(
        _fused_kernel,
        grid=(_GRID,),
        in_specs=[
            pl.BlockSpec((_R, CODE_DIM), lambda i: (i, 0)),
            const_spec((NB_CODE, CODE_DIM + 128)),
            const_spec((NB_CODE, ATTN_DIM)),
            const_spec((CODE_DIM, ATTN_DIM)),
        ],
        out_specs=[
            pl.BlockSpec((_R, CODE_DIM), lambda i: (i, 0)),
            const_spec((1, 128)),
            const_spec((1, 128)),
            const_spec((1, 128)),
            const_spec((1, 128)),
            const_spec((1, 128)),
        ],
        out_shape=[
            jax.ShapeDtypeStruct((_ROWS, CODE_DIM), f32),
            scal, scal, scal, scal, scal,
        ],
        scratch_shapes=[
            pltpu.VMEM((8, NB_CODE), f32),
            pltpu.VMEM((8, NB_CODE), f32),
            pltpu.VMEM((_R, 1), f32),
            pltpu.VMEM((1, CODE_DIM), f32),
        ],
        compiler_params=pltpu.CompilerParams(
            dimension_semantics=("arbitrary",),
        ),
        interpret=interpret,
    )(xf, cb_aug, kn, Wq)


def kernel(x, codebook, Wq, Wk):
    N, C, T = x.shape
    xf = jnp.transpose(x, (0, 2, 1)).reshape(N * T, C)
    z, ent, vq, ppl, sent, aent = _run(xf, codebook, Wq, Wk)
    x_d = jnp.transpose(z.reshape(N, T, C), (0, 2, 1))
    return (x_d, ent[0, 0], vq[0, 0], ppl[0, 0], sent[0, 0], aent[0, 0])


# speedup vs baseline: 1.3617x; 1.3617x over previous
import jax
import jax.numpy as jnp
from jax.experimental import pallas as pl
from jax.experimental.pallas import tpu as pltpu

NB_CODE = 8192
CODE_DIM = 256
ATTN_DIM = 256
ENT_T = 0.01
EPS = 1e-05

_R = 256
_ROWS = 32 * 576
_GRID = _ROWS // _R
_SCALE = float(ATTN_DIM) ** -0.5


def _keys_kernel(cb_ref, wk_ref, kn_ref):
    k = jnp.dot(cb_ref[...], wk_ref[...], preferred_element_type=jnp.float32,
                precision=None)
    kn = k / (jnp.sqrt(jnp.sum(k * k, axis=1, keepdims=True)) + 1e-6)
    kn_ref[...] = kn * _SCALE


def _fused_kernel(xf_ref, cb_ref, kn_ref, wq_ref,
                  z_ref, ent_ref, vq_ref, ppl_ref, sent_ref, aent_ref,
                  psum_ref, csum_ref, sent_acc_ref, vq_acc_ref):
    i = pl.program_id(0)

    @pl.when(i == 0)
    def _init():
        psum_ref[...] = jnp.zeros_like(psum_ref)
        csum_ref[...] = jnp.zeros_like(csum_ref)
        sent_acc_ref[...] = jnp.zeros_like(sent_acc_ref)
        vq_acc_ref[...] = jnp.zeros_like(vq_acc_ref)

    xb = xf_ref[...]
    q = jnp.dot(xb, wq_ref[...], preferred_element_type=jnp.float32,
                precision=None)
    qn = q / (jnp.sqrt(jnp.sum(q * q, axis=1, keepdims=True)) + 1e-6)
    s = jax.lax.dot_general(qn, kn_ref[...], (((1,), (1,)), ((), ())),
                            preferred_element_type=jnp.float32,
                            precision=None)
    f32 = jnp.float32
    _sub = _R // 8
    _colsum8 = lambda a: jnp.sum(a.reshape(_sub, 8, NB_CODE), axis=0)

    e = jnp.exp(s)
    denom = jnp.sum(e, axis=1, keepdims=True)
    zq = jnp.dot(e, cb_ref[...], preferred_element_type=jnp.float32,
                 precision=None) / denom
    d = zq - xb
    z_ref[...] = xb + d
    vq_acc_ref[...] += jnp.sum(d * d, axis=0, keepdims=True)

    e2 = jnp.exp(s * (1.0 / ENT_T))
    z2 = jnp.sum(e2, axis=1, keepdims=True)
    ses = jnp.sum(e2 * s, axis=1, keepdims=True)
    r2 = 1.0 / z2
    sent_acc_ref[...] += ses * (r2 * (1.0 / ENT_T)) - jnp.log(z2)
    psum_ref[...] += _colsum8(e2 * r2)

    m = jnp.max(s, axis=1, keepdims=True)
    oh = (s == m).astype(f32)
    colsum = _colsum8(oh)
    any_tie = jnp.sum(colsum) > _R + 0.5

    @pl.when(jnp.logical_not(any_tie))
    def _fast():
        csum_ref[...] += colsum

    @pl.when(any_tie)
    def _exact():
        iota = jax.lax.broadcasted_iota(jnp.int32, s.shape, 1)
        idx = jnp.min(jnp.where(s == m, iota, NB_CODE), axis=1,
                      keepdims=True)
        csum_ref[...] += _colsum8((iota == idx).astype(f32))

    @pl.when(i == _GRID - 1)
    def _fin():
        nrows = float(_ROWS)
        avg_p = jnp.sum(psum_ref[...], axis=0, keepdims=True) / nrows
        aent = -jnp.sum(avg_p * jnp.log(avg_p + EPS))
        sent = -jnp.sum(sent_acc_ref[...]) / nrows
        counts = jnp.sum(csum_ref[...], axis=0, keepdims=True)
        prob = counts / jnp.sum(counts)
        ppl = jnp.exp(-jnp.sum(prob * jnp.log(prob + 1e-7)))
        vq = jnp.sum(vq_acc_ref[...]) / (nrows * CODE_DIM)

        def bcast(v):
            return jnp.broadcast_to(jnp.reshape(v, (1, 1)), (1, 128))
        ent_ref[...] = bcast(sent - aent)
        vq_ref[...] = bcast(vq)
        ppl_ref[...] = bcast(ppl)
        sent_ref[...] = bcast(sent)
        aent_ref[...] = bcast(aent)


def _run(xf, codebook, Wq, Wk, interpret=False):
    f32 = jnp.float32
    scal = jax.ShapeDtypeStruct((1, 128), f32)
    const_spec = lambda shape: pl.BlockSpec(shape, lambda i: (0, 0))
    kn = pl.pallas_call(
        _keys_kernel,
        grid=(8,),
        in_specs=[
            pl.BlockSpec((NB_CODE // 8, CODE_DIM), lambda i: (i, 0)),
            pl.BlockSpec((CODE_DIM, ATTN_DIM), lambda i: (0, 0)),
        ],
        out_specs=pl.BlockSpec((NB_CODE // 8, ATTN_DIM), lambda i: (i, 0)),
        out_shape=jax.ShapeDtypeStruct((NB_CODE, ATTN_DIM), f32),
        interpret=interpret,
    )(codebook, Wk)
    return pl.pallas_call(
        _fused_kernel,
        grid=(_GRID,),
        in_specs=[
            pl.BlockSpec((_R, CODE_DIM), lambda i: (i, 0)),
            const_spec((NB_CODE, CODE_DIM)),
            const_spec((NB_CODE, ATTN_DIM)),
            const_spec((CODE_DIM, ATTN_DIM)),
        ],
        out_specs=[
            pl.BlockSpec((_R, CODE_DIM), lambda i: (i, 0)),
            const_spec((1, 128)),
            const_spec((1, 128)),
            const_spec((1, 128)),
            const_spec((1, 128)),
            const_spec((1, 128)),
        ],
        out_shape=[
            jax.ShapeDtypeStruct((_ROWS, CODE_DIM), f32),
            scal, scal, scal, scal, scal,
        ],
        scratch_shapes=[
            pltpu.VMEM((8, NB_CODE), f32),
            pltpu.VMEM((8, NB_CODE), f32),
            pltpu.VMEM((_R, 1), f32),
            pltpu.VMEM((1, CODE_DIM), f32),
        ],
        compiler_params=pltpu.CompilerParams(
            dimension_semantics=("arbitrary",),
        ),
        interpret=interpret,
    )(xf, codebook, kn, Wq)


def kernel(x, codebook, Wq, Wk):
    N, C, T = x.shape
    xf = jnp.transpose(x, (0, 2, 1)).reshape(N * T, C)
    z, ent, vq, ppl, sent, aent = _run(xf, codebook, Wq, Wk)
    x_d = jnp.transpose(z.reshape(N, T, C), (0, 2, 1))
    return (x_d, ent[0, 0], vq[0, 0], ppl[0, 0], sent[0, 0], aent[0, 0])
